# baseline (speedup 1.0000x reference)
import functools

import jax
import jax.numpy as jnp
from jax import lax
from jax.experimental import pallas as pl
from jax.experimental.pallas import tpu as pltpu
from jax.experimental.pallas import tpu_sc as plsc

DIM = 768
E = 1536
S = 16
B = 2
L = 4096
K = 1228
KP = 1280
T = 32
NC = KP // T
ETILE = 512
NET = E // ETILE
NROWS = B * L
NSCAT = B * KP


def _l2n(v):
    n = jnp.linalg.norm(v, axis=-1, keepdims=True)
    return v / jnp.maximum(n, 1e-12)


def _rank_body(p_ref, pt_ref, rank_ref, gid3_ref):
    b = pl.program_id(0)
    it = pl.program_id(1)
    pfull = p_ref[0, 0, :]
    pi = pt_ref[0, 0, :]
    JT = 1024
    pic = jax.lax.broadcast_in_dim(pi, (ETILE, JT), (0,))
    ii = lax.broadcasted_iota(jnp.int32, (ETILE, JT), 0) + it * ETILE
    rank = jnp.zeros((ETILE,), jnp.int32)
    for jt in range(L // JT):
        pj = jax.lax.broadcast_in_dim(pfull[jt * JT:(jt + 1) * JT],
                                      (ETILE, JT), (1,))
        jj = lax.broadcasted_iota(jnp.int32, (ETILE, JT), 1) + jt * JT
        cnt = (pj > pic) | ((pj == pic) & (jj < ii))
        rank = rank + jnp.sum(cnt.astype(jnp.int32), axis=1)
    labs = lax.broadcasted_iota(jnp.int32, (ETILE,), 0) + it * ETILE
    gid3 = jnp.where(rank < K, b * KP + rank, NSCAT + b * L + labs)
    rank_ref[0, 0, :] = rank
    gid3_ref[0, 0, :] = gid3


def _rank_call(p):
    p3 = p.reshape(B, 1, L)
    return pl.pallas_call(
        _rank_body,
        grid=(B, L // ETILE),
        in_specs=[pl.BlockSpec((1, 1, L), lambda b, it: (b, 0, 0)),
                  pl.BlockSpec((1, 1, ETILE), lambda b, it: (b, 0, it))],
        out_specs=[pl.BlockSpec((1, 1, ETILE), lambda b, it: (b, 0, it)),
                   pl.BlockSpec((1, 1, ETILE), lambda b, it: (b, 0, it))],
        out_shape=[jax.ShapeDtypeStruct((B, 1, L), jnp.int32),
                   jax.ShapeDtypeStruct((B, 1, L), jnp.int32)],
    )(p3, p3)


def _extract_body(rank_ref, gidx_ref):
    b = pl.program_id(0)
    rt = pl.program_id(1)
    rfull = rank_ref[0, 0, :]
    rr = lax.broadcasted_iota(jnp.int32, (128, L), 0) + rt * 128
    rj = jax.lax.broadcast_in_dim(rfull, (128, L), (1,))
    jj = lax.broadcasted_iota(jnp.int32, (128, L), 1)
    hit = (rj == rr)
    idx = jnp.sum(jnp.where(hit, jj, 0), axis=1)
    rglob = lax.broadcasted_iota(jnp.int32, (128,), 0) + rt * 128
    gidx = jnp.where(rglob < K, idx + b * L, 0)
    gidx_ref[0, 0, :] = gidx


def _extract_call(rank):
    return pl.pallas_call(
        _extract_body,
        grid=(B, KP // 128),
        in_specs=[pl.BlockSpec((1, 1, L), lambda b, rt: (b, 0, 0))],
        out_specs=pl.BlockSpec((1, 1, 128), lambda b, rt: (b, 0, rt)),
        out_shape=jax.ShapeDtypeStruct((B, 1, KP), jnp.int32),
    )(rank)


def _prep_body(a_ref, bp_ref, cp_ref, pw_ref, ap_ref, sc_ref):
    a = a_ref[...]
    u = jax.nn.sigmoid(bp_ref[0, :]).reshape(1, S)
    sc_ref[...] = jax.nn.sigmoid(cp_ref[...])
    eye = (lax.broadcasted_iota(jnp.int32, (S, S), 0) ==
           lax.broadcasted_iota(jnp.int32, (S, S), 1)).astype(jnp.float32)
    at = lax.dot_general(eye, a, (((1,), (1,)), ((), ())),
                         preferred_element_type=jnp.float32)
    pw_ref[0:1, :] = u
    ap_ref[0:S, :] = at

    def body(d, carry):
        pw_prev, ap_prev = carry
        pw_d = lax.dot_general(pw_prev, a, (((1,), (1,)), ((), ())),
                               preferred_element_type=jnp.float32)
        ap_d = lax.dot_general(ap_prev, a, (((1,), (1,)), ((), ())),
                               preferred_element_type=jnp.float32)
        pw_ref[pl.ds(d, 1), :] = pw_d
        ap_ref[pl.ds(d * S, S), :] = ap_d
        return pw_d, ap_d

    lax.fori_loop(1, T, body, (u, at))


def _prep_call(A, Bp, Cp):
    return pl.pallas_call(
        _prep_body,
        in_specs=[pl.BlockSpec((S, S), lambda: (0, 0)),
                  pl.BlockSpec((1, S), lambda: (0, 0)),
                  pl.BlockSpec((E, S), lambda: (0, 0))],
        out_specs=[pl.BlockSpec((T, S), lambda: (0, 0)),
                   pl.BlockSpec((T * S, S), lambda: (0, 0)),
                   pl.BlockSpec((E, S), lambda: (0, 0))],
        out_shape=[jax.ShapeDtypeStruct((T, S), jnp.float32),
                   jax.ShapeDtypeStruct((T * S, S), jnp.float32),
                   jax.ShapeDtypeStruct((E, S), jnp.float32)],
    )(A, Bp.reshape(1, S), Cp)


SSTEPS = B * NC


def _scan_body(xs_ref, xg_ref, cw_ref, toep_ref, ap3_ref, prev_ref,
               mmatt_ref, sct_ref, wout_ref, bout_ref, out_ref,
               h_ref, tail_ref):
    i = pl.program_id(0)
    c = i % NC

    @pl.when(c == 0)
    def _():
        h_ref[...] = jnp.zeros((S, E), jnp.float32)
        tail_ref[...] = jnp.zeros((8, E), jnp.float32)

    if True:
        xsb = xs_ref[...]
        w = cw_ref[...]
        tail = tail_ref[0:3, :]
        ext = jnp.concatenate([tail, xsb], axis=0)
        xc = (w[3:4, :] * xsb
              + w[2:3, :] * ext[2:T + 2, :]
              + w[1:2, :] * ext[1:T + 1, :]
              + w[0:1, :] * ext[0:T, :])
        tail_ref[0:3, :] = xsb[T - 3:T, :]
        xcb = xc.astype(jnp.bfloat16)

        out_acc = jnp.zeros((T, E), jnp.float32)
        for s in range(S):
            ys = jnp.dot(toep_ref[s], xcb, preferred_element_type=jnp.float32)
            out_acc = out_acc + ys * sct_ref[s:s + 1, :]

        h_et = h_ref[...]
        v = jnp.concatenate(
            [h_et[s:s + 1, :] * sct_ref[...] for s in range(S)], axis=0)
        out_acc = out_acc + jnp.dot(ap3_ref[...], v,
                                    preferred_element_type=jnp.float32)

        h_ref[...] = (
            jnp.dot(mmatt_ref[...], h_et, preferred_element_type=jnp.float32)
            + lax.dot_general(prev_ref[...], xc, (((0,), (0,)), ((), ())),
                              preferred_element_type=jnp.float32))

        part = jnp.dot(out_acc.astype(jnp.bfloat16), wout_ref[...],
                       preferred_element_type=jnp.float32)
        out_ref[...] = part + bout_ref[...] + xg_ref[...]

def _scan_call(xs, xg, cwT, toep, ap3, prev, mmatT, sCT, W_out, b_out):
    return pl.pallas_call(
        _scan_body,
        grid=(SSTEPS,),
        in_specs=[
            pl.BlockSpec((T, E), lambda i: (i, 0)),
            pl.BlockSpec((T, DIM), lambda i: (i, 0)),
            pl.BlockSpec((4, E), lambda i: (0, 0)),
            pl.BlockSpec((S, T, T), lambda i: (0, 0, 0)),
            pl.BlockSpec((T, S * S), lambda i: (0, 0)),
            pl.BlockSpec((T, S), lambda i: (0, 0)),
            pl.BlockSpec((S, S), lambda i: (0, 0)),
            pl.BlockSpec((S, E), lambda i: (0, 0)),
            pl.BlockSpec((E, DIM), lambda i: (0, 0)),
            pl.BlockSpec((1, DIM), lambda i: (0, 0)),
        ],
        out_specs=pl.BlockSpec((T, DIM), lambda i: (i, 0)),
        out_shape=jax.ShapeDtypeStruct((NSCAT, DIM), jnp.float32),
        scratch_shapes=[pltpu.VMEM((S, E), jnp.float32),
                        pltpu.VMEM((8, E), jnp.float32)],
        compiler_params=pltpu.CompilerParams(
            dimension_semantics=("arbitrary",)),
    )(xs, xg, cwT, toep, ap3, prev, mmatT, sCT, W_out, b_out)


def _in_gather(xp2d, x2d, gidx):
    info = plsc.get_sparse_core_info()
    nw = info.num_cores * info.num_subcores
    rows_w = NSCAT // nw
    half = rows_w // 2
    mesh = plsc.VectorSubcoreMesh(core_axis_name="c", subcore_axis_name="s")

    @functools.partial(
        pl.kernel,
        out_type=[jax.ShapeDtypeStruct((NSCAT, E), jnp.float32),
                  jax.ShapeDtypeStruct((NSCAT, DIM), jnp.float32)],
        mesh=mesh,
        scratch_types=[pltpu.VMEM((half,), jnp.int32),
                       pltpu.VMEM((half, E), jnp.float32),
                       pltpu.VMEM((half, DIM), jnp.float32),
                       pltpu.SemaphoreType.DMA,
                       pltpu.SemaphoreType.DMA],
    )
    def k(xp_hbm, x_hbm, idx_hbm, xs_hbm, xg_hbm, idx_v, rp_v, rx_v, s1, s2):
        wid = lax.axis_index("s") * info.num_cores + lax.axis_index("c")
        for r in range(2):
            base = wid * rows_w + r * half
            pltpu.sync_copy(idx_hbm.at[pl.ds(base, half)], idx_v)
            c1 = pltpu.async_copy(xp_hbm.at[idx_v], rp_v, s1)
            c2 = pltpu.async_copy(x_hbm.at[idx_v], rx_v, s2)
            c1.wait()
            c2.wait()
            pltpu.sync_copy(rp_v, xs_hbm.at[pl.ds(base, half)])
            pltpu.sync_copy(rx_v, xg_hbm.at[pl.ds(base, half)])

    return k(xp2d, x2d, gidx)


def _out_gather(table, gid3):
    info = plsc.get_sparse_core_info()
    nw = info.num_cores * info.num_subcores
    rows_w = NROWS // nw
    chunk = 64
    mesh = plsc.VectorSubcoreMesh(core_axis_name="c", subcore_axis_name="s")

    @functools.partial(
        pl.kernel,
        out_type=jax.ShapeDtypeStruct((NROWS, DIM), jnp.float32),
        mesh=mesh,
        scratch_types=[pltpu.VMEM((chunk,), jnp.int32),
                       pltpu.VMEM((chunk, DIM), jnp.float32),
                       pltpu.SemaphoreType.DMA],
    )
    def k(tab_hbm, idx_hbm, out_hbm, idx_v, rows_v, sem):
        wid = lax.axis_index("s") * info.num_cores + lax.axis_index("c")
        for r in range(rows_w // chunk):
            base = wid * rows_w + r * chunk
            pltpu.sync_copy(idx_hbm.at[pl.ds(base, chunk)], idx_v)
            pltpu.async_copy(tab_hbm.at[idx_v], rows_v, sem).wait()
            pltpu.sync_copy(rows_v, out_hbm.at[pl.ds(base, chunk)])

    return k(table, gid3)


def kernel(x, norm_w, W_in, b_in, W_out, b_out, A, Bp, Cp, conv_w):
    xn = x * jax.lax.rsqrt(jnp.mean(x * x, axis=-1, keepdims=True) + 1e-6) * norm_w
    x_proj = xn @ W_in + b_in
    center = x_proj[:, L // 2:L // 2 + 1, :]
    sim = jnp.squeeze(jnp.matmul(_l2n(x_proj), jnp.swapaxes(_l2n(center), -1, -2)), -1)
    p = jax.nn.softmax(sim, axis=-1)

    rank, gid3 = _rank_call(p)
    gidx = _extract_call(rank)

    xp2d = x_proj.reshape(NROWS, E)
    x2d = x.reshape(NROWS, DIM)
    xs, xg = _in_gather(xp2d, x2d, gidx.reshape(NSCAT))

    pw, apflat, sC = _prep_call(A, Bp, Cp)
    ap = apflat.reshape(T, S, S)
    ap3 = ap.reshape(T, S * S)
    tt = jnp.arange(T)[:, None]
    jjj = jnp.arange(T)[None, :]
    d = tt - jjj
    toep = jnp.where((d >= 0)[:, :, None],
                     jnp.take(pw, jnp.clip(d, 0, T - 1), axis=0),
                     0.0).transpose(2, 0, 1).astype(jnp.bfloat16)
    prevm = pw[::-1]
    mmatT = ap[T - 1].T
    cwT = conv_w.reshape(E, 4).T

    x_scat = _scan_call(xs, xg, cwT, toep, ap3, prevm, mmatT,
                        sC.T, W_out.astype(jnp.bfloat16), b_out.reshape(1, DIM))

    table = jnp.concatenate([x_scat, x2d], axis=0)
    out2d = _out_gather(table, gid3.reshape(NROWS))
    return out2d.reshape(B, L, DIM)


# speedup vs baseline: 1.0509x; 1.0509x over previous
import functools

import jax
import jax.numpy as jnp
from jax import lax
from jax.experimental import pallas as pl
from jax.experimental.pallas import tpu as pltpu
from jax.experimental.pallas import tpu_sc as plsc

DIM = 768
E = 1536
S = 16
B = 2
L = 4096
K = 1228
KP = 1280
T = 64
NC = KP // T
ETILE = 512
NET = E // ETILE
NROWS = B * L
NSCAT = B * KP


def _l2n(v):
    n = jnp.linalg.norm(v, axis=-1, keepdims=True)
    return v / jnp.maximum(n, 1e-12)


def _rank_body(p_ref, pt_ref, rank_ref, gid3_ref):
    b = pl.program_id(0)
    it = pl.program_id(1)
    pfull = p_ref[0, 0, :]
    pi = pt_ref[0, 0, :]
    JT = 1024
    pic = jax.lax.broadcast_in_dim(pi, (ETILE, JT), (0,))
    ii = lax.broadcasted_iota(jnp.int32, (ETILE, JT), 0) + it * ETILE
    rank = jnp.zeros((ETILE,), jnp.int32)
    for jt in range(L // JT):
        pj = jax.lax.broadcast_in_dim(pfull[jt * JT:(jt + 1) * JT],
                                      (ETILE, JT), (1,))
        jj = lax.broadcasted_iota(jnp.int32, (ETILE, JT), 1) + jt * JT
        cnt = (pj > pic) | ((pj == pic) & (jj < ii))
        rank = rank + jnp.sum(cnt.astype(jnp.int32), axis=1)
    labs = lax.broadcasted_iota(jnp.int32, (ETILE,), 0) + it * ETILE
    gid3 = jnp.where(rank < K, b * KP + rank, NSCAT + b * L + labs)
    rank_ref[0, 0, :] = rank
    gid3_ref[0, 0, :] = gid3


def _rank_call(p):
    p3 = p.reshape(B, 1, L)
    return pl.pallas_call(
        _rank_body,
        grid=(B, L // ETILE),
        in_specs=[pl.BlockSpec((1, 1, L), lambda b, it: (b, 0, 0)),
                  pl.BlockSpec((1, 1, ETILE), lambda b, it: (b, 0, it))],
        out_specs=[pl.BlockSpec((1, 1, ETILE), lambda b, it: (b, 0, it)),
                   pl.BlockSpec((1, 1, ETILE), lambda b, it: (b, 0, it))],
        out_shape=[jax.ShapeDtypeStruct((B, 1, L), jnp.int32),
                   jax.ShapeDtypeStruct((B, 1, L), jnp.int32)],
    )(p3, p3)


def _extract_body(rank_ref, gidx_ref):
    b = pl.program_id(0)
    rt = pl.program_id(1)
    rfull = rank_ref[0, 0, :]
    rr = lax.broadcasted_iota(jnp.int32, (128, L), 0) + rt * 128
    rj = jax.lax.broadcast_in_dim(rfull, (128, L), (1,))
    jj = lax.broadcasted_iota(jnp.int32, (128, L), 1)
    hit = (rj == rr)
    idx = jnp.sum(jnp.where(hit, jj, 0), axis=1)
    rglob = lax.broadcasted_iota(jnp.int32, (128,), 0) + rt * 128
    gidx = jnp.where(rglob < K, idx + b * L, 0)
    gidx_ref[0, 0, :] = gidx


def _extract_call(rank):
    return pl.pallas_call(
        _extract_body,
        grid=(B, KP // 128),
        in_specs=[pl.BlockSpec((1, 1, L), lambda b, rt: (b, 0, 0))],
        out_specs=pl.BlockSpec((1, 1, 128), lambda b, rt: (b, 0, rt)),
        out_shape=jax.ShapeDtypeStruct((B, 1, KP), jnp.int32),
    )(rank)


def _prep_body(a_ref, bp_ref, cp_ref, pw_ref, ap_ref, sc_ref):
    a = a_ref[...]
    u = jax.nn.sigmoid(bp_ref[0, :]).reshape(1, S)
    sc_ref[...] = jax.nn.sigmoid(cp_ref[...])
    eye = (lax.broadcasted_iota(jnp.int32, (S, S), 0) ==
           lax.broadcasted_iota(jnp.int32, (S, S), 1)).astype(jnp.float32)
    at = lax.dot_general(eye, a, (((1,), (1,)), ((), ())),
                         preferred_element_type=jnp.float32)
    pw_ref[0:1, :] = u
    ap_ref[0:S, :] = at

    def body(d, carry):
        pw_prev, ap_prev = carry
        pw_d = lax.dot_general(pw_prev, a, (((1,), (1,)), ((), ())),
                               preferred_element_type=jnp.float32)
        ap_d = lax.dot_general(ap_prev, a, (((1,), (1,)), ((), ())),
                               preferred_element_type=jnp.float32)
        pw_ref[pl.ds(d, 1), :] = pw_d
        ap_ref[pl.ds(d * S, S), :] = ap_d
        return pw_d, ap_d

    lax.fori_loop(1, T, body, (u, at))


def _prep_call(A, Bp, Cp):
    return pl.pallas_call(
        _prep_body,
        in_specs=[pl.BlockSpec((S, S), lambda: (0, 0)),
                  pl.BlockSpec((1, S), lambda: (0, 0)),
                  pl.BlockSpec((E, S), lambda: (0, 0))],
        out_specs=[pl.BlockSpec((T, S), lambda: (0, 0)),
                   pl.BlockSpec((T * S, S), lambda: (0, 0)),
                   pl.BlockSpec((E, S), lambda: (0, 0))],
        out_shape=[jax.ShapeDtypeStruct((T, S), jnp.float32),
                   jax.ShapeDtypeStruct((T * S, S), jnp.float32),
                   jax.ShapeDtypeStruct((E, S), jnp.float32)],
    )(A, Bp.reshape(1, S), Cp)


SSTEPS = B * NC


def _scan_body(xs_ref, xg_ref, cw_ref, toep_ref, ap3_ref, prev_ref,
               mmatt_ref, sct_ref, wout_ref, bout_ref, out_ref,
               h_ref, tail_ref):
    i = pl.program_id(0)
    c = i % NC

    @pl.when(c == 0)
    def _():
        h_ref[...] = jnp.zeros((S, E), jnp.float32)
        tail_ref[...] = jnp.zeros((8, E), jnp.float32)

    if True:
        xsb = xs_ref[...]
        w = cw_ref[...]
        tail = tail_ref[0:3, :]
        ext = jnp.concatenate([tail, xsb], axis=0)
        xc = (w[3:4, :] * xsb
              + w[2:3, :] * ext[2:T + 2, :]
              + w[1:2, :] * ext[1:T + 1, :]
              + w[0:1, :] * ext[0:T, :])
        tail_ref[0:3, :] = xsb[T - 3:T, :]
        xcb = xc.astype(jnp.bfloat16)

        out_acc = jnp.zeros((T, E), jnp.float32)
        for s in range(S):
            ys = jnp.dot(toep_ref[s], xcb, preferred_element_type=jnp.float32)
            out_acc = out_acc + ys * sct_ref[s:s + 1, :]

        h_et = h_ref[...]
        v = jnp.concatenate(
            [h_et[s:s + 1, :] * sct_ref[...] for s in range(S)], axis=0)
        out_acc = out_acc + jnp.dot(ap3_ref[...], v,
                                    preferred_element_type=jnp.float32)

        h_ref[...] = (
            jnp.dot(mmatt_ref[...], h_et, preferred_element_type=jnp.float32)
            + lax.dot_general(prev_ref[...], xc, (((0,), (0,)), ((), ())),
                              preferred_element_type=jnp.float32))

        part = jnp.dot(out_acc.astype(jnp.bfloat16), wout_ref[...],
                       preferred_element_type=jnp.float32)
        out_ref[...] = part + bout_ref[...] + xg_ref[...]

def _scan_call(xs, xg, cwT, toep, ap3, prev, mmatT, sCT, W_out, b_out):
    return pl.pallas_call(
        _scan_body,
        grid=(SSTEPS,),
        in_specs=[
            pl.BlockSpec((T, E), lambda i: (i, 0)),
            pl.BlockSpec((T, DIM), lambda i: (i, 0)),
            pl.BlockSpec((4, E), lambda i: (0, 0)),
            pl.BlockSpec((S, T, T), lambda i: (0, 0, 0)),
            pl.BlockSpec((T, S * S), lambda i: (0, 0)),
            pl.BlockSpec((T, S), lambda i: (0, 0)),
            pl.BlockSpec((S, S), lambda i: (0, 0)),
            pl.BlockSpec((S, E), lambda i: (0, 0)),
            pl.BlockSpec((E, DIM), lambda i: (0, 0)),
            pl.BlockSpec((1, DIM), lambda i: (0, 0)),
        ],
        out_specs=pl.BlockSpec((T, DIM), lambda i: (i, 0)),
        out_shape=jax.ShapeDtypeStruct((NSCAT, DIM), jnp.float32),
        scratch_shapes=[pltpu.VMEM((S, E), jnp.float32),
                        pltpu.VMEM((8, E), jnp.float32)],
        compiler_params=pltpu.CompilerParams(
            dimension_semantics=("arbitrary",)),
    )(xs, xg, cwT, toep, ap3, prev, mmatT, sCT, W_out, b_out)


def _in_gather(xp2d, x2d, gidx):
    info = plsc.get_sparse_core_info()
    nw = info.num_cores * info.num_subcores
    rows_w = NSCAT // nw
    half = 16
    nr = rows_w // half
    mesh = plsc.VectorSubcoreMesh(core_axis_name="c", subcore_axis_name="s")

    @functools.partial(
        pl.kernel,
        out_type=[jax.ShapeDtypeStruct((NSCAT, E), jnp.float32),
                  jax.ShapeDtypeStruct((NSCAT, DIM), jnp.float32)],
        mesh=mesh,
        scratch_types=[pltpu.VMEM((half,), jnp.int32),
                       pltpu.VMEM((half,), jnp.int32),
                       pltpu.VMEM((half, E), jnp.float32),
                       pltpu.VMEM((half, E), jnp.float32),
                       pltpu.VMEM((half, DIM), jnp.float32),
                       pltpu.VMEM((half, DIM), jnp.float32),
                       pltpu.SemaphoreType.DMA,
                       pltpu.SemaphoreType.DMA,
                       pltpu.SemaphoreType.DMA,
                       pltpu.SemaphoreType.DMA],
    )
    def k(xp_hbm, x_hbm, idx_hbm, xs_hbm, xg_hbm,
          ia_v, ib_v, pa_v, pb_v, xa_v, xb_v, s0, s1, s2, s3):
        wid = lax.axis_index("s") * info.num_cores + lax.axis_index("c")
        base0 = wid * rows_w
        bufs = ((ia_v, pa_v, xa_v, s0, s1), (ib_v, pb_v, xb_v, s2, s3))

        def start(r):
            iv, pv, xv, sp, sx = bufs[r % 2]
            pltpu.sync_copy(idx_hbm.at[pl.ds(base0 + r * half, half)], iv)
            c1 = pltpu.async_copy(xp_hbm.at[iv], pv, sp)
            c2 = pltpu.async_copy(x_hbm.at[iv], xv, sx)
            return c1, c2

        cs = start(0)
        for r in range(nr):
            nxt = start(r + 1) if r + 1 < nr else None
            cs[0].wait()
            cs[1].wait()
            _, pv, xv, _, _ = bufs[r % 2]
            pltpu.sync_copy(pv, xs_hbm.at[pl.ds(base0 + r * half, half)])
            pltpu.sync_copy(xv, xg_hbm.at[pl.ds(base0 + r * half, half)])
            cs = nxt

    return k(xp2d, x2d, gidx)


def _out_gather(table, gid3):
    info = plsc.get_sparse_core_info()
    nw = info.num_cores * info.num_subcores
    rows_w = NROWS // nw
    chunk = 64
    nr = rows_w // chunk
    mesh = plsc.VectorSubcoreMesh(core_axis_name="c", subcore_axis_name="s")

    @functools.partial(
        pl.kernel,
        out_type=jax.ShapeDtypeStruct((NROWS, DIM), jnp.float32),
        mesh=mesh,
        scratch_types=[pltpu.VMEM((chunk,), jnp.int32),
                       pltpu.VMEM((chunk,), jnp.int32),
                       pltpu.VMEM((chunk, DIM), jnp.float32),
                       pltpu.VMEM((chunk, DIM), jnp.float32),
                       pltpu.SemaphoreType.DMA,
                       pltpu.SemaphoreType.DMA],
    )
    def k(tab_hbm, idx_hbm, out_hbm, ia_v, ib_v, ra_v, rb_v, s0, s1):
        wid = lax.axis_index("s") * info.num_cores + lax.axis_index("c")
        base0 = wid * rows_w
        bufs = ((ia_v, ra_v, s0), (ib_v, rb_v, s1))

        def start(r):
            iv, rv, sm = bufs[r % 2]
            pltpu.sync_copy(idx_hbm.at[pl.ds(base0 + r * chunk, chunk)], iv)
            return pltpu.async_copy(tab_hbm.at[iv], rv, sm)

        c = start(0)
        for r in range(nr):
            nxt = start(r + 1) if r + 1 < nr else None
            c.wait()
            _, rv, _ = bufs[r % 2]
            pltpu.sync_copy(rv, out_hbm.at[pl.ds(base0 + r * chunk, chunk)])
            c = nxt

    return k(table, gid3)


def kernel(x, norm_w, W_in, b_in, W_out, b_out, A, Bp, Cp, conv_w):
    xn = x * jax.lax.rsqrt(jnp.mean(x * x, axis=-1, keepdims=True) + 1e-6) * norm_w
    x_proj = xn @ W_in + b_in
    center = x_proj[:, L // 2:L // 2 + 1, :]
    sim = jnp.squeeze(jnp.matmul(_l2n(x_proj), jnp.swapaxes(_l2n(center), -1, -2)), -1)
    p = jax.nn.softmax(sim, axis=-1)

    rank, gid3 = _rank_call(p)
    gidx = _extract_call(rank)

    xp2d = x_proj.reshape(NROWS, E)
    x2d = x.reshape(NROWS, DIM)
    xs, xg = _in_gather(xp2d, x2d, gidx.reshape(NSCAT))

    pw, apflat, sC = _prep_call(A, Bp, Cp)
    ap = apflat.reshape(T, S, S)
    ap3 = ap.reshape(T, S * S)
    tt = jnp.arange(T)[:, None]
    jjj = jnp.arange(T)[None, :]
    d = tt - jjj
    toep = jnp.where((d >= 0)[:, :, None],
                     jnp.take(pw, jnp.clip(d, 0, T - 1), axis=0),
                     0.0).transpose(2, 0, 1).astype(jnp.bfloat16)
    prevm = pw[::-1]
    mmatT = ap[T - 1].T
    cwT = conv_w.reshape(E, 4).T

    x_scat = _scan_call(xs, xg, cwT, toep, ap3, prevm, mmatT,
                        sC.T, W_out.astype(jnp.bfloat16), b_out.reshape(1, DIM))

    table = jnp.concatenate([x_scat, x2d], axis=0)
    out2d = _out_gather(table, gid3.reshape(NROWS))
    return out2d.reshape(B, L, DIM)
